# Initial kernel scaffold; baseline (speedup 1.0000x reference)
import functools
import math

import jax
import jax.numpy as jnp
from jax.experimental import pallas as pl
from jax.experimental.pallas import tpu as pltpu
from jax.experimental.pallas import tpu_sc as plsc

D_EMBED = 32
SCALE = math.sqrt(D_EMBED)
WINDOW = 128
LANES = 16


@functools.lru_cache(maxsize=None)
def _build(n_idx: int):
  mesh = plsc.VectorSubcoreMesh(core_axis_name="core", subcore_axis_name="subcore")

  @functools.partial(
      pl.kernel,
      out_type=jax.ShapeDtypeStruct((n_idx, D_EMBED), jnp.float32),
      mesh=mesh,
  )
  def emb_kernel(table_hbm, idx_hbm, out_hbm):
    def body(i_vmem, o_vmem):
      pltpu.sync_copy(table_hbm.at[i_vmem.at[0]], o_vmem)

      @pl.loop(0, WINDOW)
      def _(r):
        for c in range(D_EMBED // LANES):
          slc = (pl.ds(r, 1), pl.ds(c * LANES, LANES))
          o_vmem.at[slc][...] = o_vmem.at[slc][...] * SCALE

    pltpu.emit_pipeline(
        body,
        grid=(n_idx // WINDOW,),
        in_specs=[pl.BlockSpec((1, WINDOW), index_map=lambda i: (0, i))],
        out_specs=[pl.BlockSpec((WINDOW, D_EMBED), index_map=lambda i: (i, 0))],
        core_axis_name=("core", "subcore"),
        dimension_semantics=(pltpu.PARALLEL,),
    )(idx_hbm, out_hbm)

  return emb_kernel


def kernel(x, table):
  batch, hist = x.shape
  n = batch * hist
  idx = x.reshape(1, n).astype(jnp.int32)
  out = _build(n)(table, idx)
  return out.reshape(batch, hist, D_EMBED)


# speedup vs baseline: 1.0693x; 1.0693x over previous
import functools
import math

import jax
import jax.numpy as jnp
from jax.experimental import pallas as pl
from jax.experimental.pallas import tpu as pltpu
from jax.experimental.pallas import tpu_sc as plsc

D_EMBED = 32
SCALE = math.sqrt(D_EMBED)
WINDOW = 128
LANES = 16


@functools.lru_cache(maxsize=None)
def _build(n_idx: int):
  mesh = plsc.VectorSubcoreMesh(core_axis_name="core", subcore_axis_name="subcore")

  @functools.partial(
      pl.kernel,
      out_type=jax.ShapeDtypeStruct((n_idx, D_EMBED), jnp.float32),
      mesh=mesh,
      compiler_params=pltpu.CompilerParams(use_tc_tiling_on_sc=False),
  )
  def emb_kernel(table_hbm, idx_hbm, out_hbm):
    def body(i_vmem, o_vmem):
      pltpu.sync_copy(table_hbm.at[i_vmem.at[0]], o_vmem)

      @pl.loop(0, WINDOW)
      def _(r):
        for c in range(D_EMBED // LANES):
          slc = (pl.ds(r, 1), pl.ds(c * LANES, LANES))
          o_vmem.at[slc][...] = o_vmem.at[slc][...] * SCALE

    pltpu.emit_pipeline(
        body,
        grid=(n_idx // WINDOW,),
        in_specs=[pl.BlockSpec((1, WINDOW), index_map=lambda i: (0, i))],
        out_specs=[pl.BlockSpec((WINDOW, D_EMBED), index_map=lambda i: (i, 0))],
        core_axis_name=("core", "subcore"),
        dimension_semantics=(pltpu.PARALLEL,),
    )(idx_hbm, out_hbm)

  return emb_kernel


def kernel(x, table):
  batch, hist = x.shape
  n = batch * hist
  idx = x.reshape(1, n).astype(jnp.int32)
  out = _build(n)(table, idx)
  return out.reshape(batch, hist, D_EMBED)


# speedup vs baseline: 1.4752x; 1.3795x over previous
import functools
import math

import jax
import jax.numpy as jnp
from jax import lax
from jax.experimental import pallas as pl
from jax.experimental.pallas import tpu as pltpu
from jax.experimental.pallas import tpu_sc as plsc

D_EMBED = 32
SCALE = math.sqrt(D_EMBED)
LANES = 16
NC, NS = 2, 16
NW = NC * NS
SUB = 128
CHUNK = 1024
NBUF = 3
ROWS_PER_LOOP = 8


@functools.lru_cache(maxsize=None)
def _build(n_idx: int):
  b_per_w = n_idx // NW
  n_chunks = b_per_w // CHUNK
  assert n_idx == NW * n_chunks * CHUNK and CHUNK % SUB == 0 and n_chunks >= 3

  mesh = plsc.VectorSubcoreMesh(core_axis_name="core", subcore_axis_name="subcore")

  @functools.partial(
      pl.kernel,
      out_type=jax.ShapeDtypeStruct((n_idx, D_EMBED), jnp.float32),
      mesh=mesh,
      compiler_params=pltpu.CompilerParams(use_tc_tiling_on_sc=False),
      scratch_types=[
          pltpu.VMEM((b_per_w,), jnp.int32),
          pltpu.VMEM((NBUF, CHUNK, D_EMBED), jnp.float32),
          pltpu.SemaphoreType.DMA,
          pltpu.SemaphoreType.DMA,
          pltpu.SemaphoreType.DMA,
          pltpu.SemaphoreType.DMA,
          pltpu.SemaphoreType.DMA,
          pltpu.SemaphoreType.DMA,
          pltpu.SemaphoreType.DMA,
      ],
  )
  def emb_kernel(table_hbm, idx_hbm, out_hbm, idx_v, rows_v, isem,
                 g0, g1, g2, o0, o1, o2):
    gsem = [g0, g1, g2]
    osem = [o0, o1, o2]
    wid = lax.axis_index("subcore") * NC + lax.axis_index("core")
    base = wid * b_per_w

    pltpu.async_copy(idx_hbm.at[pl.ds(base, b_per_w)], idx_v, isem).wait()

    def fire_gathers(g):
      p = g % NBUF
      cps = []
      for j in range(CHUNK // SUB):
        cp = pltpu.make_async_copy(
            table_hbm.at[idx_v.at[pl.ds(g * CHUNK + j * SUB, SUB)]],
            rows_v.at[p, pl.ds(j * SUB, SUB)],
            gsem[p])
        cp.start()
        cps.append(cp)
      return cps

    def scale_chunk(p):
      @pl.loop(0, CHUNK, step=ROWS_PER_LOOP)
      def _(r):
        for i in range(ROWS_PER_LOOP):
          for c in range(D_EMBED // LANES):
            slc = (p, pl.ds(r + i, 1), pl.ds(c * LANES, LANES))
            rows_v.at[slc][...] = rows_v.at[slc][...] * SCALE

    def fire_out(g):
      p = g % NBUF
      cp = pltpu.make_async_copy(
          rows_v.at[p], out_hbm.at[pl.ds(base + g * CHUNK, CHUNK)], osem[p])
      cp.start()
      return cp

    gd = {0: fire_gathers(0), 1: fire_gathers(1)}
    od = {}
    for g in range(n_chunks):
      for cp in gd.pop(g):
        cp.wait()
      scale_chunk(g % NBUF)
      od[g] = fire_out(g)
      nxt = g + 2
      if nxt < n_chunks:
        if g >= 1:
          od.pop(g - 1).wait()
        gd[nxt] = fire_gathers(nxt)
    for g in sorted(od):
      od.pop(g).wait()

  return emb_kernel


def kernel(x, table):
  batch, hist = x.shape
  n = batch * hist
  idx = x.reshape(n).astype(jnp.int32)
  out = _build(n)(table, idx)
  return out.reshape(batch, hist, D_EMBED)
